# baseline (speedup 1.0000x reference)
import functools

import jax
import jax.numpy as jnp
from jax import lax
from jax.experimental import pallas as pl
from jax.experimental.pallas import tpu as pltpu
from jax.experimental.pallas import tpu_sc as plsc

B = 2048
D = 2048
F = 16384
K = 32
N = B * F
KSEL = K * B

NC = 2
NS = 16
NW = NC * NS
SHARD = N // NW
WIN = 16384
NWIN = SHARD // WIN
LANES = 16
WROW = 8
WCOL = WIN // WROW
ROWS_PER_W = B // NW

TF = 512

CAP = 16384
CAPL = CAP // 16


def _sc_hist_kernel(bins, match_shift, digit_shift, digit_mask, use_prefix,
                    compact=False):
    mesh = plsc.VectorSubcoreMesh(
        core_axis_name="c", subcore_axis_name="s", num_cores=NC, num_subcores=NS
    )

    stride = bins + 16
    scratch = [
        pltpu.VMEM((16, WCOL), jnp.float32),
        pltpu.VMEM((LANES * stride,), jnp.int32),
        pltpu.VMEM((bins,), jnp.int32),
        pltpu.VMEM((16,), jnp.int32),
        pltpu.SemaphoreType.DMA,
        pltpu.SemaphoreType.DMA,
    ]
    if compact:
        scratch.append(pltpu.VMEM((CAP,), jnp.float32))
        scratch.append(pltpu.VMEM((16,), jnp.int32))
        out_type = (
            jax.ShapeDtypeStruct((NW, bins), jnp.int32),
            jax.ShapeDtypeStruct((NW, CAP), jnp.float32),
            jax.ShapeDtypeStruct((NW, 16), jnp.int32),
        )
    else:
        out_type = jax.ShapeDtypeStruct((NW, bins), jnp.int32)

    def body(scores_hbm, pfx_hbm, out_hbm, *rest):
        if compact:
            cand_hbm, cnt_hbm, buf, hist, acc, pfx_v, sem0, sem1, candv, cnt_v = rest
        else:
            buf, hist, acc, pfx_v, sem0, sem1 = rest
        cid = lax.axis_index("c")
        sid = lax.axis_index("s")
        wid = cid * NS + sid
        base_row = wid * ROWS_PER_W

        if use_prefix:
            pltpu.sync_copy(pfx_hbm, pfx_v)
            pfx = pfx_v[...]
        else:
            pfx = jnp.zeros((16,), jnp.int32)

        zeros16 = jnp.zeros((16,), jnp.int32)
        ones16 = jnp.ones((16,), jnp.int32)
        lane = lax.iota(jnp.int32, 16)
        laneoff = lane * (stride + 1)

        @plsc.parallel_loop(0, LANES * stride // 16, unroll=8)
        def _(i):
            hist[pl.ds(i * 16, 16)] = zeros16

        def proc16(v, cnt):
            bits = lax.bitcast_convert_type(v, jnp.int32)
            if use_prefix:
                m = lax.shift_right_logical(bits, match_shift) == pfx
            else:
                m = None
            d = lax.shift_right_logical(bits, digit_shift)
            if digit_mask is not None:
                d = jnp.bitwise_and(d, digit_mask)
            idx = laneoff + d
            plsc.addupdate_scatter(hist, [idx], ones16, mask=m)
            if compact:
                slot = lane + 16 * jnp.minimum(cnt, CAPL - 1)
                plsc.store_scatter(candv, [slot], v, mask=m)
                cnt = cnt + jnp.where(m, 1, 0).astype(jnp.int32)
            return cnt

        def process(bref, cnt):
            @plsc.parallel_loop(0, WCOL // 16, unroll=2, carry=cnt)
            def inner(i, c):
                for row in range(WROW):
                    c = proc16(bref[row, pl.ds(i * 16, 16)], c)
                return c

            return inner

        CPR = F // WCOL

        def win_src(w):
            r0 = base_row + (w // CPR) * WROW
            c0 = (w % CPR) * WCOL
            return scores_hbm.at[pl.ds(r0, WROW), pl.ds(c0, WCOL)]

        buf0 = buf.at[pl.ds(0, WROW)]
        buf1 = buf.at[pl.ds(WROW, WROW)]
        pltpu.async_copy(win_src(0), buf0, sem0)

        def wbody(j, cnt):
            pltpu.async_copy(win_src(2 * j + 1), buf1, sem1)
            pltpu.make_async_copy(win_src(2 * j), buf0, sem0).wait()
            cnt = process(buf0, cnt)

            @pl.when(j < NWIN // 2 - 1)
            def _():
                pltpu.async_copy(win_src(2 * j + 2), buf0, sem0)

            pltpu.make_async_copy(win_src(2 * j + 1), buf1, sem1).wait()
            cnt = process(buf1, cnt)
            return cnt

        total = lax.fori_loop(0, NWIN // 2, wbody, jnp.zeros((16,), jnp.int32))
        if compact:
            cnt_v[...] = total
            pltpu.sync_copy(cnt_v, cnt_hbm.at[wid])
            pltpu.sync_copy(candv, cand_hbm.at[wid])

        @plsc.parallel_loop(0, bins // 16, unroll=2)
        def _(j):
            s = hist[pl.ds(j * 16, 16)]
            for l in range(1, LANES):
                s = s + hist[pl.ds(l * (stride + 1) + j * 16, 16)]
            acc[pl.ds(j * 16, 16)] = s
        pltpu.sync_copy(acc, out_hbm.at[wid])

    return functools.partial(
        pl.kernel,
        out_type=out_type,
        mesh=mesh,
        scratch_types=scratch,
        compiler_params=pltpu.CompilerParams(
            needs_layout_passes=False, use_tc_tiling_on_sc=True
        ),
    )(body)


def _sc_cand_kernel(bins, match_shift, digit_mask):
    mesh = plsc.VectorSubcoreMesh(
        core_axis_name="c", subcore_axis_name="s", num_cores=NC, num_subcores=NS
    )
    stride = bins + 16
    scratch = [
        pltpu.VMEM((CAP,), jnp.float32),
        pltpu.VMEM((LANES * stride,), jnp.int32),
        pltpu.VMEM((bins,), jnp.int32),
        pltpu.VMEM((16,), jnp.int32),
        pltpu.VMEM((16,), jnp.int32),
    ]

    def body(cand_hbm, cnt_hbm, pfx_hbm, out_hbm, cbuf, hist, acc, pfx_v, cnt_v):
        cid = lax.axis_index("c")
        sid = lax.axis_index("s")
        wid = cid * NS + sid
        pltpu.sync_copy(cand_hbm.at[wid], cbuf)
        pltpu.sync_copy(cnt_hbm.at[wid], cnt_v)
        pltpu.sync_copy(pfx_hbm, pfx_v)
        pfx = pfx_v[...]
        mycnt = cnt_v[...]

        zeros16 = jnp.zeros((16,), jnp.int32)
        ones16 = jnp.ones((16,), jnp.int32)
        lane = lax.iota(jnp.int32, 16)
        laneoff = lane * (stride + 1)

        @plsc.parallel_loop(0, LANES * stride // 16, unroll=8)
        def _(i):
            hist[pl.ds(i * 16, 16)] = zeros16

        @plsc.parallel_loop(0, CAP // 16, unroll=8)
        def _(i):
            v = cbuf[pl.ds(i * 16, 16)]
            bits = lax.bitcast_convert_type(v, jnp.int32)
            pos = jnp.zeros((16,), jnp.int32) + i
            m = jnp.logical_and(
                pos < mycnt, lax.shift_right_logical(bits, match_shift) == pfx
            )
            d = jnp.bitwise_and(bits, digit_mask)
            plsc.addupdate_scatter(hist, [laneoff + d], ones16, mask=m)

        @plsc.parallel_loop(0, bins // 16, unroll=2)
        def _(j):
            s = hist[pl.ds(j * 16, 16)]
            for l in range(1, LANES):
                s = s + hist[pl.ds(l * (stride + 1) + j * 16, 16)]
            acc[pl.ds(j * 16, 16)] = s

        pltpu.sync_copy(acc, out_hbm.at[wid])

    return functools.partial(
        pl.kernel,
        out_type=jax.ShapeDtypeStruct((NW, bins), jnp.int32),
        mesh=mesh,
        scratch_types=scratch,
        compiler_params=pltpu.CompilerParams(needs_layout_passes=False),
    )(body)


def _tc_scan_kernel(bins, shift, first, last, with_ovf=False):
    def body(*refs):
        if first:
            (hist_ref, pfxo_ref, ko_ref) = refs
            k = jnp.int32(KSEL)
            pfx = jnp.int32(0)
        else:
            if with_ovf:
                (hist_ref, pfxi_ref, ki_ref, cnt_ref, *outs) = refs
            else:
                (hist_ref, pfxi_ref, ki_ref, *outs) = refs
            k = jnp.max(ki_ref[...])
            pfx = jnp.max(pfxi_ref[...])
            if last:
                (tau_ref,) = outs
            elif with_ovf:
                (pfxo_ref, ko_ref, ovf_ref) = outs
                ovf_ref[...] = jnp.full(
                    (1, 16),
                    jnp.where(jnp.max(cnt_ref[...]) > CAPL, 1, 0),
                    jnp.int32,
                )
            else:
                (pfxo_ref, ko_ref) = outs

        cnt = jnp.sum(hist_ref[...], axis=0, keepdims=True)
        ge = cnt
        s = 1
        while s < bins:
            ge = ge + jnp.concatenate(
                [ge[:, s:], jnp.zeros((1, s), jnp.int32)], axis=1
            )
            s *= 2
        d_iota = lax.broadcasted_iota(jnp.int32, (1, bins), 1)
        valid = ge >= k
        T = jnp.max(jnp.where(valid, d_iota, -1))
        sel = d_iota == T
        cntT = jnp.max(jnp.where(sel, cnt, 0))
        geT = jnp.max(jnp.where(sel, ge, 0))
        k_next = k - (geT - cntT)
        new_pfx = jnp.bitwise_or(lax.shift_left(pfx, shift), T)
        if last:
            tau_ref[...] = jnp.full(
                (1, 16), lax.bitcast_convert_type(new_pfx, jnp.float32)
            )
        else:
            pfxo_ref[...] = jnp.full((1, 16), new_pfx, jnp.int32)
            ko_ref[...] = jnp.full((1, 16), k_next, jnp.int32)

    if last:
        outs = jax.ShapeDtypeStruct((1, 16), jnp.float32)
    elif with_ovf:
        outs = (
            jax.ShapeDtypeStruct((1, 16), jnp.int32),
            jax.ShapeDtypeStruct((1, 16), jnp.int32),
            jax.ShapeDtypeStruct((1, 16), jnp.int32),
        )
    else:
        outs = (
            jax.ShapeDtypeStruct((1, 16), jnp.int32),
            jax.ShapeDtypeStruct((1, 16), jnp.int32),
        )
    return pl.pallas_call(body, out_shape=outs)


def _norms_kernel(w_dec):
    def body(w_ref, out_ref):
        out_ref[...] = jnp.sqrt(jnp.sum(w_ref[...] * w_ref[...], axis=1))

    return pl.pallas_call(
        body,
        grid=(F // 512,),
        in_specs=[pl.BlockSpec((512, D), lambda i: (i, 0))],
        out_specs=pl.BlockSpec((512,), lambda i: (i,)),
        out_shape=jax.ShapeDtypeStruct((F,), jnp.float32),
    )(w_dec)


def _encode_kernel(x, w_enc, b_enc, b_dec, norms_1f):
    def body(x_ref, w_ref, benc_ref, bdec_ref, nrm_ref, acts_ref, scores_ref):
        xc = x_ref[...] - bdec_ref[...]
        acts = jnp.maximum(
            jnp.dot(xc, w_ref[...], preferred_element_type=jnp.float32)
            + benc_ref[...],
            0.0,
        )
        acts_ref[...] = acts
        scores_ref[...] = acts * nrm_ref[...]

    return pl.pallas_call(
        body,
        grid=(F // TF,),
        in_specs=[
            pl.BlockSpec((B, D), lambda i: (0, 0)),
            pl.BlockSpec((D, TF), lambda i: (0, i)),
            pl.BlockSpec((TF,), lambda i: (i,)),
            pl.BlockSpec((D,), lambda i: (0,)),
            pl.BlockSpec((1, TF), lambda i: (0, i)),
        ],
        out_specs=[
            pl.BlockSpec((B, TF), lambda i: (0, i)),
            pl.BlockSpec((B, TF), lambda i: (0, i)),
        ],
        out_shape=[
            jax.ShapeDtypeStruct((B, F), jnp.float32),
            jax.ShapeDtypeStruct((B, F), jnp.float32),
        ],
    )(x, w_enc, b_enc, b_dec, norms_1f)


def _decode_kernel(acts, norms_1f, tau, w_dec, b_dec):
    TFD = 512

    def body(acts_ref, nrm_ref, tau_ref, w_ref, bdec_ref, sparse_ref, recon_ref):
        t = jnp.max(tau_ref[...])
        scores = acts_ref[...] * nrm_ref[...]
        sp = jnp.where(scores >= t, acts_ref[...], 0.0)
        sparse_ref[...] = sp

        @pl.when(pl.program_id(0) == 0)
        def _():
            recon_ref[...] = jnp.zeros((B, D), jnp.float32) + bdec_ref[...]

        recon_ref[...] += jnp.dot(
            sp, w_ref[...], preferred_element_type=jnp.float32
        )

    return pl.pallas_call(
        body,
        grid=(F // TFD,),
        in_specs=[
            pl.BlockSpec((B, TFD), lambda i: (0, i)),
            pl.BlockSpec((1, TFD), lambda i: (0, i)),
            pl.BlockSpec((1, 16), lambda i: (0, 0)),
            pl.BlockSpec((TFD, D), lambda i: (i, 0)),
            pl.BlockSpec((D,), lambda i: (0,)),
        ],
        out_specs=[
            pl.BlockSpec((B, TFD), lambda i: (0, i)),
            pl.BlockSpec((B, D), lambda i: (0, 0)),
        ],
        out_shape=[
            jax.ShapeDtypeStruct((B, F), jnp.float32),
            jax.ShapeDtypeStruct((B, D), jnp.float32),
        ],
    )(acts, norms_1f, tau, w_dec, b_dec)


def kernel(x_BD, W_encoder_DF, b_encoder_F, W_decoder_FD, b_decoder_D):
    norms_F = _norms_kernel(W_decoder_FD)
    norms_1f = norms_F.reshape(1, F)

    acts, scores = _encode_kernel(
        x_BD, W_encoder_DF, b_encoder_F, b_decoder_D, norms_1f
    )

    h1 = _sc_hist_kernel(4096, 0, 19, None, False)(
        scores, jnp.zeros((16,), jnp.int32)
    )
    p1, k1 = _tc_scan_kernel(4096, 12, True, False)(h1)
    h2, cand, ccnt = _sc_hist_kernel(4096, 19, 7, 0xFFF, True, compact=True)(
        scores, p1.reshape(16)
    )
    p2, k2, ovf = _tc_scan_kernel(4096, 12, False, False, with_ovf=True)(
        h2, p1, k1, ccnt
    )
    p2v = p2.reshape(16)
    h3 = lax.cond(
        jnp.max(ovf) > 0,
        lambda: _sc_hist_kernel(128, 7, 0, 0x7F, True)(scores, p2v),
        lambda: _sc_cand_kernel(128, 7, 0x7F)(cand, ccnt, p2v),
    )
    tau = _tc_scan_kernel(128, 7, False, True)(h3, p2, k2)

    sparse, recon = _decode_kernel(acts, norms_1f, tau, W_decoder_FD, b_decoder_D)
    return recon, sparse, acts


# speedup vs baseline: 1.3346x; 1.3346x over previous
import functools

import jax
import jax.numpy as jnp
from jax import lax
from jax.experimental import pallas as pl
from jax.experimental.pallas import tpu as pltpu
from jax.experimental.pallas import tpu_sc as plsc

B = 2048
D = 2048
F = 16384
K = 32
N = B * F
KSEL = K * B

NC = 2
NS = 16
NW = NC * NS
SHARD = N // NW
WIN = 16384
NWIN = SHARD // WIN
LANES = 16

TF = 512

CAP = 16384
CAPL = CAP // 16


def _sc_hist_kernel(bins, match_shift, digit_shift, digit_mask, use_prefix,
                    compact=False):
    mesh = plsc.VectorSubcoreMesh(
        core_axis_name="c", subcore_axis_name="s", num_cores=NC, num_subcores=NS
    )

    stride = bins + 16
    scratch = [
        pltpu.VMEM((2 * WIN,), jnp.float32),
        pltpu.VMEM((LANES * stride,), jnp.int32),
        pltpu.VMEM((bins,), jnp.int32),
        pltpu.VMEM((16,), jnp.int32),
        pltpu.SemaphoreType.DMA,
        pltpu.SemaphoreType.DMA,
    ]
    if compact:
        scratch.append(pltpu.VMEM((CAP,), jnp.float32))
        scratch.append(pltpu.VMEM((16,), jnp.int32))
        out_type = (
            jax.ShapeDtypeStruct((NW, bins), jnp.int32),
            jax.ShapeDtypeStruct((NW, CAP), jnp.float32),
            jax.ShapeDtypeStruct((NW, 16), jnp.int32),
        )
    else:
        out_type = jax.ShapeDtypeStruct((NW, bins), jnp.int32)

    def body(scores_hbm, pfx_hbm, out_hbm, *rest):
        if compact:
            cand_hbm, cnt_hbm, buf, hist, acc, pfx_v, sem0, sem1, candv, cnt_v = rest
        else:
            buf, hist, acc, pfx_v, sem0, sem1 = rest
        cid = lax.axis_index("c")
        sid = lax.axis_index("s")
        wid = cid * NS + sid
        base = wid * SHARD

        if use_prefix:
            pltpu.sync_copy(pfx_hbm, pfx_v)
            pfx = pfx_v[...]
        else:
            pfx = jnp.zeros((16,), jnp.int32)

        zeros16 = jnp.zeros((16,), jnp.int32)
        ones16 = jnp.ones((16,), jnp.int32)
        lane = lax.iota(jnp.int32, 16)
        laneoff = lane * (stride + 1)

        @plsc.parallel_loop(0, LANES * stride // 16, unroll=8)
        def _(i):
            hist[pl.ds(i * 16, 16)] = zeros16

        def proc16(bref, off, cnt):
            v = bref[pl.ds(off, 16)]
            bits = lax.bitcast_convert_type(v, jnp.int32)
            if use_prefix:
                m = lax.shift_right_logical(bits, match_shift) == pfx
            else:
                m = None
            d = lax.shift_right_logical(bits, digit_shift)
            if digit_mask is not None:
                d = jnp.bitwise_and(d, digit_mask)
            idx = laneoff + d
            plsc.addupdate_scatter(hist, [idx], ones16, mask=m)
            if compact:
                slot = lane + 16 * jnp.minimum(cnt, CAPL - 1)
                plsc.store_scatter(candv, [slot], v, mask=m)
                cnt = cnt + jnp.where(m, 1, 0).astype(jnp.int32)
            return cnt

        def process(bref, cnt):
            @plsc.parallel_loop(0, WIN // 16, unroll=8, carry=cnt)
            def inner(i, c):
                return proc16(bref, i * 16, c)

            return inner

        def win_src(w):
            return scores_hbm.at[pl.ds(base + w * WIN, WIN)]

        buf0 = buf.at[pl.ds(0, WIN)]
        buf1 = buf.at[pl.ds(WIN, WIN)]
        pltpu.async_copy(win_src(0), buf0, sem0)

        def wbody(j, cnt):
            pltpu.async_copy(win_src(2 * j + 1), buf1, sem1)
            pltpu.make_async_copy(win_src(2 * j), buf0, sem0).wait()
            cnt = process(buf0, cnt)

            @pl.when(j < NWIN // 2 - 1)
            def _():
                pltpu.async_copy(win_src(2 * j + 2), buf0, sem0)

            pltpu.make_async_copy(win_src(2 * j + 1), buf1, sem1).wait()
            cnt = process(buf1, cnt)
            return cnt

        total = lax.fori_loop(0, NWIN // 2, wbody, jnp.zeros((16,), jnp.int32))
        if compact:
            cnt_v[...] = total
            pltpu.sync_copy(cnt_v, cnt_hbm.at[wid])
            pltpu.sync_copy(candv, cand_hbm.at[wid])

        @plsc.parallel_loop(0, bins // 16, unroll=2)
        def _(j):
            s = hist[pl.ds(j * 16, 16)]
            for l in range(1, LANES):
                s = s + hist[pl.ds(l * (stride + 1) + j * 16, 16)]
            acc[pl.ds(j * 16, 16)] = s
        pltpu.sync_copy(acc, out_hbm.at[wid])

    return functools.partial(
        pl.kernel,
        out_type=out_type,
        mesh=mesh,
        scratch_types=scratch,
        compiler_params=pltpu.CompilerParams(needs_layout_passes=False),
    )(body)


def _sc_cand_kernel(bins, match_shift, digit_mask):
    mesh = plsc.VectorSubcoreMesh(
        core_axis_name="c", subcore_axis_name="s", num_cores=NC, num_subcores=NS
    )
    stride = bins + 16
    scratch = [
        pltpu.VMEM((CAP,), jnp.float32),
        pltpu.VMEM((LANES * stride,), jnp.int32),
        pltpu.VMEM((bins,), jnp.int32),
        pltpu.VMEM((16,), jnp.int32),
        pltpu.VMEM((16,), jnp.int32),
    ]

    def body(cand_hbm, cnt_hbm, pfx_hbm, out_hbm, cbuf, hist, acc, pfx_v, cnt_v):
        cid = lax.axis_index("c")
        sid = lax.axis_index("s")
        wid = cid * NS + sid
        pltpu.sync_copy(cand_hbm.at[wid], cbuf)
        pltpu.sync_copy(cnt_hbm.at[wid], cnt_v)
        pltpu.sync_copy(pfx_hbm, pfx_v)
        pfx = pfx_v[...]
        mycnt = cnt_v[...]

        zeros16 = jnp.zeros((16,), jnp.int32)
        ones16 = jnp.ones((16,), jnp.int32)
        lane = lax.iota(jnp.int32, 16)
        laneoff = lane * (stride + 1)

        @plsc.parallel_loop(0, LANES * stride // 16, unroll=8)
        def _(i):
            hist[pl.ds(i * 16, 16)] = zeros16

        @plsc.parallel_loop(0, CAP // 16, unroll=8)
        def _(i):
            v = cbuf[pl.ds(i * 16, 16)]
            bits = lax.bitcast_convert_type(v, jnp.int32)
            pos = jnp.zeros((16,), jnp.int32) + i
            m = jnp.logical_and(
                pos < mycnt, lax.shift_right_logical(bits, match_shift) == pfx
            )
            d = jnp.bitwise_and(bits, digit_mask)
            plsc.addupdate_scatter(hist, [laneoff + d], ones16, mask=m)

        @plsc.parallel_loop(0, bins // 16, unroll=2)
        def _(j):
            s = hist[pl.ds(j * 16, 16)]
            for l in range(1, LANES):
                s = s + hist[pl.ds(l * (stride + 1) + j * 16, 16)]
            acc[pl.ds(j * 16, 16)] = s

        pltpu.sync_copy(acc, out_hbm.at[wid])

    return functools.partial(
        pl.kernel,
        out_type=jax.ShapeDtypeStruct((NW, bins), jnp.int32),
        mesh=mesh,
        scratch_types=scratch,
        compiler_params=pltpu.CompilerParams(needs_layout_passes=False),
    )(body)


def _tc_scan_kernel(bins, shift, first, last, with_ovf=False):
    def body(*refs):
        if first:
            (hist_ref, pfxo_ref, ko_ref) = refs
            k = jnp.int32(KSEL)
            pfx = jnp.int32(0)
        else:
            if with_ovf:
                (hist_ref, pfxi_ref, ki_ref, cnt_ref, *outs) = refs
            else:
                (hist_ref, pfxi_ref, ki_ref, *outs) = refs
            k = jnp.max(ki_ref[...])
            pfx = jnp.max(pfxi_ref[...])
            if last:
                (tau_ref,) = outs
            elif with_ovf:
                (pfxo_ref, ko_ref, ovf_ref) = outs
                ovf_ref[...] = jnp.full(
                    (1, 16),
                    jnp.where(jnp.max(cnt_ref[...]) > CAPL, 1, 0),
                    jnp.int32,
                )
            else:
                (pfxo_ref, ko_ref) = outs

        cnt = jnp.sum(hist_ref[...], axis=0, keepdims=True)
        ge = cnt
        s = 1
        while s < bins:
            ge = ge + jnp.concatenate(
                [ge[:, s:], jnp.zeros((1, s), jnp.int32)], axis=1
            )
            s *= 2
        d_iota = lax.broadcasted_iota(jnp.int32, (1, bins), 1)
        valid = ge >= k
        T = jnp.max(jnp.where(valid, d_iota, -1))
        sel = d_iota == T
        cntT = jnp.max(jnp.where(sel, cnt, 0))
        geT = jnp.max(jnp.where(sel, ge, 0))
        k_next = k - (geT - cntT)
        new_pfx = jnp.bitwise_or(lax.shift_left(pfx, shift), T)
        if last:
            tau_ref[...] = jnp.full(
                (1, 16), lax.bitcast_convert_type(new_pfx, jnp.float32)
            )
        else:
            pfxo_ref[...] = jnp.full((1, 16), new_pfx, jnp.int32)
            ko_ref[...] = jnp.full((1, 16), k_next, jnp.int32)

    if last:
        outs = jax.ShapeDtypeStruct((1, 16), jnp.float32)
    elif with_ovf:
        outs = (
            jax.ShapeDtypeStruct((1, 16), jnp.int32),
            jax.ShapeDtypeStruct((1, 16), jnp.int32),
            jax.ShapeDtypeStruct((1, 16), jnp.int32),
        )
    else:
        outs = (
            jax.ShapeDtypeStruct((1, 16), jnp.int32),
            jax.ShapeDtypeStruct((1, 16), jnp.int32),
        )
    return pl.pallas_call(body, out_shape=outs)


def _norms_kernel(w_dec):
    def body(w_ref, out_ref):
        out_ref[...] = jnp.sqrt(jnp.sum(w_ref[...] * w_ref[...], axis=1))

    return pl.pallas_call(
        body,
        grid=(F // 512,),
        in_specs=[pl.BlockSpec((512, D), lambda i: (i, 0))],
        out_specs=pl.BlockSpec((512,), lambda i: (i,)),
        out_shape=jax.ShapeDtypeStruct((F,), jnp.float32),
    )(w_dec)


def _encode_kernel(x, w_enc, b_enc, b_dec, norms_1f):
    def body(x_ref, w_ref, benc_ref, bdec_ref, nrm_ref, acts_ref, scores_ref):
        xc = x_ref[...] - bdec_ref[...]
        acts = jnp.maximum(
            jnp.dot(xc, w_ref[...], preferred_element_type=jnp.float32)
            + benc_ref[...],
            0.0,
        )
        acts_ref[...] = acts
        scores_ref[...] = acts * nrm_ref[...]

    return pl.pallas_call(
        body,
        grid=(F // TF,),
        in_specs=[
            pl.BlockSpec((B, D), lambda i: (0, 0)),
            pl.BlockSpec((D, TF), lambda i: (0, i)),
            pl.BlockSpec((TF,), lambda i: (i,)),
            pl.BlockSpec((D,), lambda i: (0,)),
            pl.BlockSpec((1, TF), lambda i: (0, i)),
        ],
        out_specs=[
            pl.BlockSpec((B, TF), lambda i: (0, i)),
            pl.BlockSpec((B, TF), lambda i: (0, i)),
        ],
        out_shape=[
            jax.ShapeDtypeStruct((B, F), jnp.float32),
            jax.ShapeDtypeStruct((B, F), jnp.float32),
        ],
    )(x, w_enc, b_enc, b_dec, norms_1f)


def _decode_kernel(acts, norms_1f, tau, w_dec, b_dec):
    TFD = 512

    def body(acts_ref, nrm_ref, tau_ref, w_ref, bdec_ref, sparse_ref, recon_ref):
        t = jnp.max(tau_ref[...])
        scores = acts_ref[...] * nrm_ref[...]
        sp = jnp.where(scores >= t, acts_ref[...], 0.0)
        sparse_ref[...] = sp

        @pl.when(pl.program_id(0) == 0)
        def _():
            recon_ref[...] = jnp.zeros((B, D), jnp.float32) + bdec_ref[...]

        recon_ref[...] += jnp.dot(
            sp, w_ref[...], preferred_element_type=jnp.float32
        )

    return pl.pallas_call(
        body,
        grid=(F // TFD,),
        in_specs=[
            pl.BlockSpec((B, TFD), lambda i: (0, i)),
            pl.BlockSpec((1, TFD), lambda i: (0, i)),
            pl.BlockSpec((1, 16), lambda i: (0, 0)),
            pl.BlockSpec((TFD, D), lambda i: (i, 0)),
            pl.BlockSpec((D,), lambda i: (0,)),
        ],
        out_specs=[
            pl.BlockSpec((B, TFD), lambda i: (0, i)),
            pl.BlockSpec((B, D), lambda i: (0, 0)),
        ],
        out_shape=[
            jax.ShapeDtypeStruct((B, F), jnp.float32),
            jax.ShapeDtypeStruct((B, D), jnp.float32),
        ],
    )(acts, norms_1f, tau, w_dec, b_dec)


def kernel(x_BD, W_encoder_DF, b_encoder_F, W_decoder_FD, b_decoder_D):
    norms_F = _norms_kernel(W_decoder_FD)
    norms_1f = norms_F.reshape(1, F)

    acts, scores = _encode_kernel(
        x_BD, W_encoder_DF, b_encoder_F, b_decoder_D, norms_1f
    )
    scores_flat = scores.reshape(N)

    h1 = _sc_hist_kernel(4096, 0, 19, None, False)(
        scores_flat, jnp.zeros((16,), jnp.int32)
    )
    p1, k1 = _tc_scan_kernel(4096, 12, True, False)(h1)
    h2, cand, ccnt = _sc_hist_kernel(4096, 19, 7, 0xFFF, True, compact=True)(
        scores_flat, p1.reshape(16)
    )
    p2, k2, ovf = _tc_scan_kernel(4096, 12, False, False, with_ovf=True)(
        h2, p1, k1, ccnt
    )
    p2v = p2.reshape(16)
    h3 = lax.cond(
        jnp.max(ovf) > 0,
        lambda: _sc_hist_kernel(128, 7, 0, 0x7F, True)(scores_flat, p2v),
        lambda: _sc_cand_kernel(128, 7, 0x7F)(cand, ccnt, p2v),
    )
    tau = _tc_scan_kernel(128, 7, False, True)(h3, p2, k2)

    sparse, recon = _decode_kernel(acts, norms_1f, tau, W_decoder_FD, b_decoder_D)
    return recon, sparse, acts
